# Initial kernel scaffold; baseline (speedup 1.0000x reference)
import jax
import jax.numpy as jnp
from jax.experimental import pallas as pl


def kernel(x, edge_index, W1, b1, W2, b2):
    raise NotImplementedError("write your pallas kernel here")


# speedup vs baseline: 21.7069x; 21.7069x over previous
import functools

import jax
import jax.numpy as jnp
from jax import lax
from jax.experimental import pallas as pl
from jax.experimental.pallas import tpu as pltpu
from jax.experimental.pallas import tpu_sc as plsc

N = 10000
E = 320000
D_IN = 128
D_HID = 64
N_CLS = 32

NC = 2
NS = 16
NW = NC * NS
CHUNK = 128
C = 80
E_PAD = NW * C * CHUNK
NACC = NW * 320
DUMMY = N
STAGE_T = 10
HS_PT = N // STAGE_T
ACC_PT = NACC // NS
DEG_W = 16

_mesh = plsc.VectorSubcoreMesh(core_axis_name="c", subcore_axis_name="s")
_sc_params = pltpu.CompilerParams(use_tc_tiling_on_sc=False)


def _deg_body(dst_hbm, ones_hbm, zeros_hbm, out_hbm, didx, onesv, deg_sh):
  cid = lax.axis_index("c")
  sid = lax.axis_index("s")
  wid = cid * NS + sid
  pltpu.sync_copy(ones_hbm, onesv)
  pltpu.sync_copy(zeros_hbm.at[pl.ds(sid * ACC_PT, ACC_PT)],
                  deg_sh.at[pl.ds(sid * ACC_PT, ACC_PT)])
  plsc.subcore_barrier()

  def step(j, carry):
    pltpu.sync_copy(dst_hbm.at[wid * C + j], didx)
    pltpu.sync_copy(onesv, deg_sh.at[didx.at[0]], add=True)
    return carry

  lax.fori_loop(0, C, step, 0)
  plsc.subcore_barrier()
  pltpu.sync_copy(deg_sh.at[pl.ds(sid * ACC_PT, ACC_PT)],
                  out_hbm.at[cid, pl.ds(sid * ACC_PT, ACC_PT)])


_deg_kernel = functools.partial(
    pl.kernel,
    _deg_body,
    out_type=jax.ShapeDtypeStruct((NC, NACC, DEG_W), jnp.float32),
    mesh=_mesh,
    compiler_params=_sc_params,
    scratch_types=[
        pltpu.VMEM((1, CHUNK), jnp.int32),
        pltpu.VMEM((CHUNK, DEG_W), jnp.float32),
        pltpu.VMEM_SHARED((NACC, DEG_W), jnp.float32),
    ],
)


def _agg_body(d, hs_hbm, src_hbm, dst_hbm, zeros_hbm, out_hbm,
              sidx, didx, msgs, gsem, hs_sh, acc_sh):
  cid = lax.axis_index("c")
  sid = lax.axis_index("s")
  wid = cid * NS + sid
  @pl.when(sid < STAGE_T)
  def _():
    pltpu.sync_copy(hs_hbm.at[pl.ds(sid * HS_PT, HS_PT)],
                    hs_sh.at[pl.ds(sid * HS_PT, HS_PT)])

  pltpu.sync_copy(zeros_hbm.at[pl.ds(sid * ACC_PT, ACC_PT)],
                  acc_sh.at[pl.ds(sid * ACC_PT, ACC_PT)])
  plsc.subcore_barrier()

  def step(j, carry):
    pltpu.sync_copy(src_hbm.at[wid * C + j], sidx)
    pltpu.sync_copy(dst_hbm.at[wid * C + j], didx)
    pltpu.async_copy(hs_sh.at[sidx.at[0]], msgs, gsem).wait()
    pltpu.sync_copy(msgs, acc_sh.at[didx.at[0]], add=True)
    return carry

  lax.fori_loop(0, C, step, 0)
  plsc.subcore_barrier()
  pltpu.sync_copy(acc_sh.at[pl.ds(sid * ACC_PT, ACC_PT)],
                  out_hbm.at[cid, pl.ds(sid * ACC_PT, ACC_PT)])


def _make_agg(d):
  return functools.partial(
      pl.kernel,
      functools.partial(_agg_body, d),
      out_type=jax.ShapeDtypeStruct((NC, NACC, d), jnp.float32),
      mesh=_mesh,
      compiler_params=_sc_params,
      scratch_types=[
          pltpu.VMEM((1, CHUNK), jnp.int32),
          pltpu.VMEM((1, CHUNK), jnp.int32),
          pltpu.VMEM((CHUNK, d), jnp.float32),
          pltpu.SemaphoreType.DMA,
          pltpu.VMEM_SHARED((N, d), jnp.float32),
          pltpu.VMEM_SHARED((NACC, d), jnp.float32),
      ],
  )


_agg64 = _make_agg(D_HID)
_agg32 = _make_agg(N_CLS)


_RB = 1000


def _tc1_body(degw_ref, x_ref, w1_ref, hs1_ref, dinv_ref):
  deg = degw_ref[0, :, :] + degw_ref[1, :, :] + 1.0
  dinv = lax.rsqrt(deg)
  dinv_ref[...] = dinv
  h = jnp.dot(x_ref[...], w1_ref[...], preferred_element_type=jnp.float32)
  hs1_ref[...] = h * dinv[:, 0:1]


def _tc1(degw, x, w1):
  return pl.pallas_call(
      _tc1_body,
      grid=(N // _RB,),
      in_specs=[
          pl.BlockSpec((NC, _RB, DEG_W), lambda j: (0, j, 0)),
          pl.BlockSpec((_RB, D_IN), lambda j: (j, 0)),
          pl.BlockSpec((D_IN, D_HID), lambda j: (0, 0)),
      ],
      out_specs=[
          pl.BlockSpec((_RB, D_HID), lambda j: (j, 0)),
          pl.BlockSpec((_RB, DEG_W), lambda j: (j, 0)),
      ],
      out_shape=[
          jax.ShapeDtypeStruct((N, D_HID), jnp.float32),
          jax.ShapeDtypeStruct((N, DEG_W), jnp.float32),
      ],
  )(degw, x, w1)


def _tc2_body(acc_ref, hs1_ref, dinv_ref, b1_ref, w2_ref, hs2_ref):
  s = acc_ref[0, :, :] + acc_ref[1, :, :] + hs1_ref[...]
  dinv = dinv_ref[:, 0:1]
  t = s * dinv + b1_ref[...]
  r = jnp.maximum(t, 0.0)
  h2 = jnp.dot(r, w2_ref[...], preferred_element_type=jnp.float32)
  hs2_ref[...] = h2 * dinv


def _tc2(acc1, hs1, dinv, b1, w2):
  return pl.pallas_call(
      _tc2_body,
      grid=(N // _RB,),
      in_specs=[
          pl.BlockSpec((NC, _RB, D_HID), lambda j: (0, j, 0)),
          pl.BlockSpec((_RB, D_HID), lambda j: (j, 0)),
          pl.BlockSpec((_RB, DEG_W), lambda j: (j, 0)),
          pl.BlockSpec((1, D_HID), lambda j: (0, 0)),
          pl.BlockSpec((D_HID, N_CLS), lambda j: (0, 0)),
      ],
      out_specs=pl.BlockSpec((_RB, N_CLS), lambda j: (j, 0)),
      out_shape=jax.ShapeDtypeStruct((N, N_CLS), jnp.float32),
  )(acc1, hs1, dinv, b1, w2)


def _tc3_body(acc_ref, hs2_ref, dinv_ref, b2_ref, out_ref):
  s = acc_ref[0, :, :] + acc_ref[1, :, :] + hs2_ref[...]
  t = s * dinv_ref[:, 0:1] + b2_ref[...]
  m = jnp.max(t, axis=1, keepdims=True)
  e = jnp.exp(t - m)
  lse = jnp.log(jnp.sum(e, axis=1, keepdims=True))
  out_ref[...] = t - m - lse


def _tc3(acc2, hs2, dinv, b2):
  return pl.pallas_call(
      _tc3_body,
      grid=(N // _RB,),
      in_specs=[
          pl.BlockSpec((NC, _RB, N_CLS), lambda j: (0, j, 0)),
          pl.BlockSpec((_RB, N_CLS), lambda j: (j, 0)),
          pl.BlockSpec((_RB, DEG_W), lambda j: (j, 0)),
          pl.BlockSpec((1, N_CLS), lambda j: (0, 0)),
      ],
      out_specs=pl.BlockSpec((_RB, N_CLS), lambda j: (j, 0)),
      out_shape=jax.ShapeDtypeStruct((N, N_CLS), jnp.float32),
  )(acc2, hs2, dinv, b2)


@jax.jit
def kernel(x, edge_index, W1, b1, W2, b2):
  pad = E_PAD - E
  src = jnp.concatenate([edge_index[0], jnp.zeros((pad,), jnp.int32)])
  dst = jnp.concatenate([edge_index[1], jnp.full((pad,), DUMMY, jnp.int32)])
  src2d = src.reshape(NW * C, 1, CHUNK)
  dst2d = dst.reshape(NW * C, 1, CHUNK)

  ones_w = jnp.ones((CHUNK, DEG_W), jnp.float32)
  zeros_w = jnp.zeros((NACC, DEG_W), jnp.float32)
  zeros64 = jnp.zeros((NACC, D_HID), jnp.float32)
  zeros32 = jnp.zeros((NACC, N_CLS), jnp.float32)

  degw = _deg_kernel()(dst2d, ones_w, zeros_w)
  hs1, dinv = _tc1(degw, x, W1)
  acc1 = _agg64()(hs1, src2d, dst2d, zeros64)
  hs2 = _tc2(acc1, hs1, dinv, b1.reshape(1, D_HID), W2)
  acc2 = _agg32()(hs2, src2d, dst2d, zeros32)
  return _tc3(acc2, hs2, dinv, b2.reshape(1, N_CLS))


# speedup vs baseline: 37.5553x; 1.7301x over previous
import functools

import jax
import jax.numpy as jnp
from jax import lax
from jax.experimental import pallas as pl
from jax.experimental.pallas import tpu as pltpu
from jax.experimental.pallas import tpu_sc as plsc

N = 10000
E = 320000
D_IN = 128
D_HID = 64
N_CLS = 32

NC = 2
NS = 16
NW = NC * NS
CHUNK = 128
C = 80
E_PAD = NW * C * CHUNK
NACC = NW * 320
DUMMY = N
STAGE_T = 10
HS_PT = N // STAGE_T
ACC_PT = NACC // NS
DEG_W = 16

_mesh = plsc.VectorSubcoreMesh(core_axis_name="c", subcore_axis_name="s")
_sc_params = pltpu.CompilerParams(use_tc_tiling_on_sc=False)


def _deg_body(dst_hbm, ones_hbm, zeros_hbm, out_hbm, dstv, onesv, ssem, deg_sh):
  cid = lax.axis_index("c")
  sid = lax.axis_index("s")
  wid = cid * NS + sid
  pltpu.sync_copy(ones_hbm, onesv)
  pltpu.sync_copy(zeros_hbm.at[pl.ds(sid * ACC_PT, ACC_PT)],
                  deg_sh.at[pl.ds(sid * ACC_PT, ACC_PT)])
  pltpu.sync_copy(dst_hbm.at[pl.ds(wid * C, C)], dstv)
  plsc.subcore_barrier()

  def fire(j, carry):
    pltpu.async_copy(onesv, deg_sh.at[dstv.at[j]], ssem, add=True)
    return carry

  def drain(j, carry):
    pltpu.make_async_copy(onesv, deg_sh.at[dstv.at[0]], ssem).wait()
    return carry

  lax.fori_loop(0, C, fire, 0)
  lax.fori_loop(0, C, drain, 0)
  plsc.subcore_barrier()
  pltpu.sync_copy(deg_sh.at[pl.ds(sid * ACC_PT, ACC_PT)],
                  out_hbm.at[cid, pl.ds(sid * ACC_PT, ACC_PT)])


_deg_kernel = functools.partial(
    pl.kernel,
    _deg_body,
    out_type=jax.ShapeDtypeStruct((NC, NACC, DEG_W), jnp.float32),
    mesh=_mesh,
    compiler_params=_sc_params,
    scratch_types=[
        pltpu.VMEM((C, CHUNK), jnp.int32),
        pltpu.VMEM((CHUNK, DEG_W), jnp.float32),
        pltpu.SemaphoreType.DMA,
        pltpu.VMEM_SHARED((NACC, DEG_W), jnp.float32),
    ],
)


def _agg_body(d, hs_hbm, src_hbm, dst_hbm, zeros_hbm, out_hbm,
              srcv, dstv, m0, m1, sem0, sem1, hs_sh, acc_sh):
  cid = lax.axis_index("c")
  sid = lax.axis_index("s")
  wid = cid * NS + sid
  @pl.when(sid < STAGE_T)
  def _():
    pltpu.sync_copy(hs_hbm.at[pl.ds(sid * HS_PT, HS_PT)],
                    hs_sh.at[pl.ds(sid * HS_PT, HS_PT)])

  pltpu.sync_copy(zeros_hbm.at[pl.ds(sid * ACC_PT, ACC_PT)],
                  acc_sh.at[pl.ds(sid * ACC_PT, ACC_PT)])
  pltpu.sync_copy(src_hbm.at[pl.ds(wid * C, C)], srcv)
  pltpu.sync_copy(dst_hbm.at[pl.ds(wid * C, C)], dstv)
  plsc.subcore_barrier()

  pltpu.async_copy(hs_sh.at[srcv.at[0]], m0, sem0)

  def step(k, carry):
    pltpu.make_async_copy(hs_sh.at[srcv.at[0]], m0, sem0).wait()
    pltpu.async_copy(hs_sh.at[srcv.at[2 * k + 1]], m1, sem1)
    pltpu.sync_copy(m0, acc_sh.at[dstv.at[2 * k]], add=True)
    pltpu.make_async_copy(hs_sh.at[srcv.at[0]], m1, sem1).wait()

    @pl.when(k < C // 2 - 1)
    def _():
      pltpu.async_copy(hs_sh.at[srcv.at[2 * k + 2]], m0, sem0)

    pltpu.sync_copy(m1, acc_sh.at[dstv.at[2 * k + 1]], add=True)
    return carry

  lax.fori_loop(0, C // 2, step, 0)
  plsc.subcore_barrier()
  pltpu.sync_copy(acc_sh.at[pl.ds(sid * ACC_PT, ACC_PT)],
                  out_hbm.at[cid, pl.ds(sid * ACC_PT, ACC_PT)])


def _make_agg(d):
  return functools.partial(
      pl.kernel,
      functools.partial(_agg_body, d),
      out_type=jax.ShapeDtypeStruct((NC, NACC, d), jnp.float32),
      mesh=_mesh,
      compiler_params=_sc_params,
      scratch_types=[
          pltpu.VMEM((C, CHUNK), jnp.int32),
          pltpu.VMEM((C, CHUNK), jnp.int32),
          pltpu.VMEM((CHUNK, d), jnp.float32),
          pltpu.VMEM((CHUNK, d), jnp.float32),
          pltpu.SemaphoreType.DMA,
          pltpu.SemaphoreType.DMA,
          pltpu.VMEM_SHARED((N, d), jnp.float32),
          pltpu.VMEM_SHARED((NACC, d), jnp.float32),
      ],
  )


_agg64 = _make_agg(D_HID)
_agg32 = _make_agg(N_CLS)


_RB = 1000


def _tc1_body(degw_ref, x_ref, w1_ref, hs1_ref, dinv_ref):
  deg = degw_ref[0, :, :] + degw_ref[1, :, :] + 1.0
  dinv = lax.rsqrt(deg)
  dinv_ref[...] = dinv
  h = jnp.dot(x_ref[...], w1_ref[...], preferred_element_type=jnp.float32)
  hs1_ref[...] = h * dinv[:, 0:1]


def _tc1(degw, x, w1):
  return pl.pallas_call(
      _tc1_body,
      grid=(N // _RB,),
      in_specs=[
          pl.BlockSpec((NC, _RB, DEG_W), lambda j: (0, j, 0)),
          pl.BlockSpec((_RB, D_IN), lambda j: (j, 0)),
          pl.BlockSpec((D_IN, D_HID), lambda j: (0, 0)),
      ],
      out_specs=[
          pl.BlockSpec((_RB, D_HID), lambda j: (j, 0)),
          pl.BlockSpec((_RB, DEG_W), lambda j: (j, 0)),
      ],
      out_shape=[
          jax.ShapeDtypeStruct((N, D_HID), jnp.float32),
          jax.ShapeDtypeStruct((N, DEG_W), jnp.float32),
      ],
  )(degw, x, w1)


def _tc2_body(acc_ref, hs1_ref, dinv_ref, b1_ref, w2_ref, hs2_ref):
  s = acc_ref[0, :, :] + acc_ref[1, :, :] + hs1_ref[...]
  dinv = dinv_ref[:, 0:1]
  t = s * dinv + b1_ref[...]
  r = jnp.maximum(t, 0.0)
  h2 = jnp.dot(r, w2_ref[...], preferred_element_type=jnp.float32)
  hs2_ref[...] = h2 * dinv


def _tc2(acc1, hs1, dinv, b1, w2):
  return pl.pallas_call(
      _tc2_body,
      grid=(N // _RB,),
      in_specs=[
          pl.BlockSpec((NC, _RB, D_HID), lambda j: (0, j, 0)),
          pl.BlockSpec((_RB, D_HID), lambda j: (j, 0)),
          pl.BlockSpec((_RB, DEG_W), lambda j: (j, 0)),
          pl.BlockSpec((1, D_HID), lambda j: (0, 0)),
          pl.BlockSpec((D_HID, N_CLS), lambda j: (0, 0)),
      ],
      out_specs=pl.BlockSpec((_RB, N_CLS), lambda j: (j, 0)),
      out_shape=jax.ShapeDtypeStruct((N, N_CLS), jnp.float32),
  )(acc1, hs1, dinv, b1, w2)


def _tc3_body(acc_ref, hs2_ref, dinv_ref, b2_ref, out_ref):
  s = acc_ref[0, :, :] + acc_ref[1, :, :] + hs2_ref[...]
  t = s * dinv_ref[:, 0:1] + b2_ref[...]
  m = jnp.max(t, axis=1, keepdims=True)
  e = jnp.exp(t - m)
  lse = jnp.log(jnp.sum(e, axis=1, keepdims=True))
  out_ref[...] = t - m - lse


def _tc3(acc2, hs2, dinv, b2):
  return pl.pallas_call(
      _tc3_body,
      grid=(N // _RB,),
      in_specs=[
          pl.BlockSpec((NC, _RB, N_CLS), lambda j: (0, j, 0)),
          pl.BlockSpec((_RB, N_CLS), lambda j: (j, 0)),
          pl.BlockSpec((_RB, DEG_W), lambda j: (j, 0)),
          pl.BlockSpec((1, N_CLS), lambda j: (0, 0)),
      ],
      out_specs=pl.BlockSpec((_RB, N_CLS), lambda j: (j, 0)),
      out_shape=jax.ShapeDtypeStruct((N, N_CLS), jnp.float32),
  )(acc2, hs2, dinv, b2)


@jax.jit
def kernel(x, edge_index, W1, b1, W2, b2):
  pad = E_PAD - E
  src = jnp.concatenate([edge_index[0], jnp.zeros((pad,), jnp.int32)])
  dst = jnp.concatenate([edge_index[1], jnp.full((pad,), DUMMY, jnp.int32)])
  src2d = src.reshape(NW * C, CHUNK)
  dst2d = dst.reshape(NW * C, CHUNK)

  ones_w = jnp.ones((CHUNK, DEG_W), jnp.float32)
  zeros_w = jnp.zeros((NACC, DEG_W), jnp.float32)
  zeros64 = jnp.zeros((NACC, D_HID), jnp.float32)
  zeros32 = jnp.zeros((NACC, N_CLS), jnp.float32)

  degw = _deg_kernel()(dst2d, ones_w, zeros_w)
  hs1, dinv = _tc1(degw, x, W1)
  acc1 = _agg64()(hs1, src2d, dst2d, zeros64)
  hs2 = _tc2(acc1, hs1, dinv, b1.reshape(1, D_HID), W2)
  acc2 = _agg32()(hs2, src2d, dst2d, zeros32)
  return _tc3(acc2, hs2, dinv, b2.reshape(1, N_CLS))


# speedup vs baseline: 41.2263x; 1.0977x over previous
import functools

import jax
import jax.numpy as jnp
from jax import lax
from jax.experimental import pallas as pl
from jax.experimental.pallas import tpu as pltpu
from jax.experimental.pallas import tpu_sc as plsc

N = 10000
E = 320000
D_IN = 128
D_HID = 64
N_CLS = 32

NC = 2
NS = 16
NW = NC * NS
CHUNK = 128
C = 81
E_PAD = NW * C * CHUNK
NACC = 10112
DUMMY = N
STAGE_T = 10
HS_PT = N // STAGE_T
ACC_PT = NACC // NS
DEG_W = 8

_mesh = plsc.VectorSubcoreMesh(core_axis_name="c", subcore_axis_name="s")
_sc_params = pltpu.CompilerParams(use_tc_tiling_on_sc=False)


def _deg_body(dst_hbm, ones_hbm, zeros_hbm, out_hbm, dstv, onesv, ssem, deg_sh):
  cid = lax.axis_index("c")
  sid = lax.axis_index("s")
  wid = cid * NS + sid
  pltpu.sync_copy(ones_hbm, onesv)
  pltpu.sync_copy(zeros_hbm.at[pl.ds(sid * ACC_PT, ACC_PT)],
                  deg_sh.at[pl.ds(sid * ACC_PT, ACC_PT)])
  pltpu.sync_copy(dst_hbm.at[pl.ds(wid * C, C)], dstv)
  plsc.subcore_barrier()

  def fire(j, carry):
    pltpu.async_copy(onesv, deg_sh.at[dstv.at[j]], ssem, add=True)
    return carry

  def drain(j, carry):
    pltpu.make_async_copy(onesv, deg_sh.at[dstv.at[0]], ssem).wait()
    return carry

  lax.fori_loop(0, C, fire, 0)
  lax.fori_loop(0, C, drain, 0)
  plsc.subcore_barrier()
  pltpu.sync_copy(deg_sh.at[pl.ds(sid * ACC_PT, ACC_PT)],
                  out_hbm.at[cid, pl.ds(sid * ACC_PT, ACC_PT)])


_deg_kernel = functools.partial(
    pl.kernel,
    _deg_body,
    out_type=jax.ShapeDtypeStruct((NC, NACC), jnp.float32),
    mesh=_mesh,
    compiler_params=_sc_params,
    scratch_types=[
        pltpu.VMEM((C, CHUNK), jnp.int32),
        pltpu.VMEM((CHUNK,), jnp.float32),
        pltpu.SemaphoreType.DMA,
        pltpu.VMEM_SHARED((NACC,), jnp.float32),
    ],
)


def _agg_body(d, hs_hbm, src_hbm, dst_hbm, zeros_hbm, out_hbm,
              srcv, dstv, m0, m1, m2,
              g0, g1, g2, s0, s1, s2, hs_sh, acc_sh):
  cid = lax.axis_index("c")
  sid = lax.axis_index("s")
  wid = cid * NS + sid
  @pl.when(sid < STAGE_T)
  def _():
    pltpu.sync_copy(hs_hbm.at[pl.ds(sid * HS_PT, HS_PT)],
                    hs_sh.at[pl.ds(sid * HS_PT, HS_PT)])

  pltpu.sync_copy(zeros_hbm.at[pl.ds(sid * ACC_PT, ACC_PT)],
                  acc_sh.at[pl.ds(sid * ACC_PT, ACC_PT)])
  pltpu.sync_copy(src_hbm.at[pl.ds(wid * C, C)], srcv)
  pltpu.sync_copy(dst_hbm.at[pl.ds(wid * C, C)], dstv)
  plsc.subcore_barrier()

  bufs = (m0, m1, m2)
  gsems = (g0, g1, g2)
  ssems = (s0, s1, s2)

  def _fire_g(j, t):
    pltpu.async_copy(hs_sh.at[srcv.at[j]], bufs[t], gsems[t])

  def _wait_g(t):
    pltpu.make_async_copy(hs_sh.at[srcv.at[0]], bufs[t], gsems[t]).wait()

  def _fire_s(j, t):
    pltpu.async_copy(bufs[t], acc_sh.at[dstv.at[j]], ssems[t], add=True)

  def _wait_s(t):
    pltpu.make_async_copy(bufs[t], acc_sh.at[dstv.at[0]], ssems[t]).wait()

  _fire_g(0, 0)
  _fire_g(1, 1)

  def step(k, carry):
    for t in range(3):
      j = 3 * k + t
      _wait_g(t)
      _fire_s(j, t)
      t2 = (t + 2) % 3

      @pl.when(j + 2 < C)
      def _():
        @pl.when(j >= 1)
        def _():
          _wait_s(t2)

        _fire_g(j + 2, t2)

    return carry

  lax.fori_loop(0, C // 3, step, 0)
  for t in range(3):
    _wait_s(t)
  plsc.subcore_barrier()
  pltpu.sync_copy(acc_sh.at[pl.ds(sid * ACC_PT, ACC_PT)],
                  out_hbm.at[cid, pl.ds(sid * ACC_PT, ACC_PT)])


def _make_agg(d):
  return functools.partial(
      pl.kernel,
      functools.partial(_agg_body, d),
      out_type=jax.ShapeDtypeStruct((NC, NACC, d), jnp.float32),
      mesh=_mesh,
      compiler_params=_sc_params,
      scratch_types=[
          pltpu.VMEM((C, CHUNK), jnp.int32),
          pltpu.VMEM((C, CHUNK), jnp.int32),
          pltpu.VMEM((CHUNK, d), jnp.float32),
          pltpu.VMEM((CHUNK, d), jnp.float32),
          pltpu.VMEM((CHUNK, d), jnp.float32),
          pltpu.SemaphoreType.DMA,
          pltpu.SemaphoreType.DMA,
          pltpu.SemaphoreType.DMA,
          pltpu.SemaphoreType.DMA,
          pltpu.SemaphoreType.DMA,
          pltpu.SemaphoreType.DMA,
          pltpu.VMEM_SHARED((N, d), jnp.float32),
          pltpu.VMEM_SHARED((NACC, d), jnp.float32),
      ],
  )


_agg64 = _make_agg(D_HID)
_agg32 = _make_agg(N_CLS)


_RB = 1024
_GRID = -(-N // _RB)


def _tc1_body(degw_ref, x_ref, w1_ref, hs1_ref, dinv_ref):
  deg = degw_ref[0, :] + degw_ref[1, :] + 1.0
  dinv = lax.rsqrt(deg)[:, None]
  dinv_ref[...] = jnp.broadcast_to(dinv, (dinv.shape[0], DEG_W))
  h = jnp.dot(x_ref[...], w1_ref[...], preferred_element_type=jnp.float32)
  hs1_ref[...] = h * dinv


def _tc1(degw, x, w1):
  return pl.pallas_call(
      _tc1_body,
      grid=(_GRID,),
      in_specs=[
          pl.BlockSpec((NC, _RB), lambda j: (0, j)),
          pl.BlockSpec((_RB, D_IN), lambda j: (j, 0)),
          pl.BlockSpec((D_IN, D_HID), lambda j: (0, 0)),
      ],
      out_specs=[
          pl.BlockSpec((_RB, D_HID), lambda j: (j, 0)),
          pl.BlockSpec((_RB, DEG_W), lambda j: (j, 0)),
      ],
      out_shape=[
          jax.ShapeDtypeStruct((N, D_HID), jnp.float32),
          jax.ShapeDtypeStruct((N, DEG_W), jnp.float32),
      ],
  )(degw, x, w1)


def _tc2_body(acc_ref, hs1_ref, dinv_ref, b1_ref, w2_ref, hs2_ref):
  s = acc_ref[0, :, :] + acc_ref[1, :, :] + hs1_ref[...]
  dinv = dinv_ref[:, 0:1]
  t = s * dinv + b1_ref[...]
  r = jnp.maximum(t, 0.0)
  h2 = jnp.dot(r, w2_ref[...], preferred_element_type=jnp.float32)
  hs2_ref[...] = h2 * dinv


def _tc2(acc1, hs1, dinv, b1, w2):
  return pl.pallas_call(
      _tc2_body,
      grid=(_GRID,),
      in_specs=[
          pl.BlockSpec((NC, _RB, D_HID), lambda j: (0, j, 0)),
          pl.BlockSpec((_RB, D_HID), lambda j: (j, 0)),
          pl.BlockSpec((_RB, DEG_W), lambda j: (j, 0)),
          pl.BlockSpec((1, D_HID), lambda j: (0, 0)),
          pl.BlockSpec((D_HID, N_CLS), lambda j: (0, 0)),
      ],
      out_specs=pl.BlockSpec((_RB, N_CLS), lambda j: (j, 0)),
      out_shape=jax.ShapeDtypeStruct((N, N_CLS), jnp.float32),
  )(acc1, hs1, dinv, b1, w2)


def _tc3_body(acc_ref, hs2_ref, dinv_ref, b2_ref, out_ref):
  s = acc_ref[0, :, :] + acc_ref[1, :, :] + hs2_ref[...]
  t = s * dinv_ref[:, 0:1] + b2_ref[...]
  m = jnp.max(t, axis=1, keepdims=True)
  e = jnp.exp(t - m)
  lse = jnp.log(jnp.sum(e, axis=1, keepdims=True))
  out_ref[...] = t - m - lse


def _tc3(acc2, hs2, dinv, b2):
  return pl.pallas_call(
      _tc3_body,
      grid=(_GRID,),
      in_specs=[
          pl.BlockSpec((NC, _RB, N_CLS), lambda j: (0, j, 0)),
          pl.BlockSpec((_RB, N_CLS), lambda j: (j, 0)),
          pl.BlockSpec((_RB, DEG_W), lambda j: (j, 0)),
          pl.BlockSpec((1, N_CLS), lambda j: (0, 0)),
      ],
      out_specs=pl.BlockSpec((_RB, N_CLS), lambda j: (j, 0)),
      out_shape=jax.ShapeDtypeStruct((N, N_CLS), jnp.float32),
  )(acc2, hs2, dinv, b2)


@jax.jit
def kernel(x, edge_index, W1, b1, W2, b2):
  pad = E_PAD - E
  src = jnp.concatenate([edge_index[0], jnp.zeros((pad,), jnp.int32)])
  dst = jnp.concatenate([edge_index[1], jnp.full((pad,), DUMMY, jnp.int32)])
  src2d = src.reshape(NW * C, CHUNK)
  dst2d = dst.reshape(NW * C, CHUNK)

  ones_w = jnp.ones((CHUNK,), jnp.float32)
  zeros_w = jnp.zeros((NACC,), jnp.float32)
  zeros64 = jnp.zeros((NACC, D_HID), jnp.float32)
  zeros32 = jnp.zeros((NACC, N_CLS), jnp.float32)

  degw = _deg_kernel()(dst2d, ones_w, zeros_w)
  hs1, dinv = _tc1(degw, x, W1)
  acc1 = _agg64()(hs1, src2d, dst2d, zeros64)
  hs2 = _tc2(acc1, hs1, dinv, b1.reshape(1, D_HID), W2)
  acc2 = _agg32()(hs2, src2d, dst2d, zeros32)
  return _tc3(acc2, hs2, dinv, b2.reshape(1, N_CLS))


# speedup vs baseline: 42.1769x; 1.0231x over previous
import functools

import jax
import jax.numpy as jnp
from jax import lax
from jax.experimental import pallas as pl
from jax.experimental.pallas import tpu as pltpu
from jax.experimental.pallas import tpu_sc as plsc

N = 10000
E = 320000
D_IN = 128
D_HID = 64
N_CLS = 32

NC = 2
NS = 16
NW = NC * NS
CHUNK = 128
C = 81
E_PAD = NW * C * CHUNK
NACC = 10112
DUMMY = N
STAGE_T = 10
HS_PT = N // STAGE_T
ACC_PT = NACC // NS
DEG_W = 8

_mesh = plsc.VectorSubcoreMesh(core_axis_name="c", subcore_axis_name="s")
_sc_params = pltpu.CompilerParams(use_tc_tiling_on_sc=False)


def _deg_body(dst_hbm, ones_hbm, zeros_hbm, out_hbm, dstv, onesv, ssem, deg_sh):
  cid = lax.axis_index("c")
  sid = lax.axis_index("s")
  wid = cid * NS + sid
  pltpu.async_copy(ones_hbm, onesv, ssem)
  pltpu.async_copy(zeros_hbm.at[pl.ds(sid * ACC_PT, ACC_PT)],
                   deg_sh.at[pl.ds(sid * ACC_PT, ACC_PT)], ssem)
  pltpu.async_copy(dst_hbm.at[pl.ds(wid * C, C)], dstv, ssem)
  pltpu.make_async_copy(ones_hbm, onesv, ssem).wait()
  pltpu.make_async_copy(zeros_hbm.at[pl.ds(sid * ACC_PT, ACC_PT)],
                        deg_sh.at[pl.ds(sid * ACC_PT, ACC_PT)], ssem).wait()
  pltpu.make_async_copy(dst_hbm.at[pl.ds(wid * C, C)], dstv, ssem).wait()
  plsc.subcore_barrier()

  def fire(j, carry):
    pltpu.async_copy(onesv, deg_sh.at[dstv.at[j]], ssem, add=True)
    return carry

  def drain(j, carry):
    pltpu.make_async_copy(onesv, deg_sh.at[dstv.at[0]], ssem).wait()
    return carry

  lax.fori_loop(0, C, fire, 0)
  lax.fori_loop(0, C, drain, 0)
  plsc.subcore_barrier()
  pltpu.sync_copy(deg_sh.at[pl.ds(sid * ACC_PT, ACC_PT)],
                  out_hbm.at[cid, pl.ds(sid * ACC_PT, ACC_PT)])


_deg_kernel = functools.partial(
    pl.kernel,
    _deg_body,
    out_type=jax.ShapeDtypeStruct((NC, NACC), jnp.float32),
    mesh=_mesh,
    compiler_params=_sc_params,
    scratch_types=[
        pltpu.VMEM((C, CHUNK), jnp.int32),
        pltpu.VMEM((CHUNK,), jnp.float32),
        pltpu.SemaphoreType.DMA,
        pltpu.VMEM_SHARED((NACC,), jnp.float32),
    ],
)


def _agg_body(d, hs_hbm, src_hbm, dst_hbm, zeros_hbm, out_hbm,
              srcv, dstv, m0, m1, m2,
              g0, g1, g2, s0, s1, s2, hs_sh, acc_sh):
  cid = lax.axis_index("c")
  sid = lax.axis_index("s")
  wid = cid * NS + sid
  @pl.when(sid < STAGE_T)
  def _():
    pltpu.async_copy(hs_hbm.at[pl.ds(sid * HS_PT, HS_PT)],
                     hs_sh.at[pl.ds(sid * HS_PT, HS_PT)], g0)

  pltpu.async_copy(zeros_hbm.at[pl.ds(sid * ACC_PT, ACC_PT)],
                   acc_sh.at[pl.ds(sid * ACC_PT, ACC_PT)], g0)
  pltpu.async_copy(src_hbm.at[pl.ds(wid * C, C)], srcv, g0)
  pltpu.async_copy(dst_hbm.at[pl.ds(wid * C, C)], dstv, g0)

  @pl.when(sid < STAGE_T)
  def _():
    pltpu.make_async_copy(hs_hbm.at[pl.ds(sid * HS_PT, HS_PT)],
                          hs_sh.at[pl.ds(sid * HS_PT, HS_PT)], g0).wait()

  pltpu.make_async_copy(zeros_hbm.at[pl.ds(sid * ACC_PT, ACC_PT)],
                        acc_sh.at[pl.ds(sid * ACC_PT, ACC_PT)], g0).wait()
  pltpu.make_async_copy(src_hbm.at[pl.ds(wid * C, C)], srcv, g0).wait()
  pltpu.make_async_copy(dst_hbm.at[pl.ds(wid * C, C)], dstv, g0).wait()
  plsc.subcore_barrier()

  bufs = (m0, m1, m2)
  gsems = (g0, g1, g2)
  ssems = (s0, s1, s2)

  def _fire_g(j, t):
    pltpu.async_copy(hs_sh.at[srcv.at[j]], bufs[t], gsems[t])

  def _wait_g(t):
    pltpu.make_async_copy(hs_sh.at[srcv.at[0]], bufs[t], gsems[t]).wait()

  def _fire_s(j, t):
    pltpu.async_copy(bufs[t], acc_sh.at[dstv.at[j]], ssems[t], add=True)

  def _wait_s(t):
    pltpu.make_async_copy(bufs[t], acc_sh.at[dstv.at[0]], ssems[t]).wait()

  _fire_g(0, 0)
  _fire_g(1, 1)

  def step(k, carry):
    for t in range(3):
      j = 3 * k + t
      _wait_g(t)
      _fire_s(j, t)
      t2 = (t + 2) % 3

      @pl.when(j + 2 < C)
      def _():
        @pl.when(j >= 1)
        def _():
          _wait_s(t2)

        _fire_g(j + 2, t2)

    return carry

  lax.fori_loop(0, C // 3, step, 0)
  for t in range(3):
    _wait_s(t)
  plsc.subcore_barrier()
  pltpu.sync_copy(acc_sh.at[pl.ds(sid * ACC_PT, ACC_PT)],
                  out_hbm.at[cid, pl.ds(sid * ACC_PT, ACC_PT)])


def _make_agg(d):
  return functools.partial(
      pl.kernel,
      functools.partial(_agg_body, d),
      out_type=jax.ShapeDtypeStruct((NC, NACC, d), jnp.float32),
      mesh=_mesh,
      compiler_params=_sc_params,
      scratch_types=[
          pltpu.VMEM((C, CHUNK), jnp.int32),
          pltpu.VMEM((C, CHUNK), jnp.int32),
          pltpu.VMEM((CHUNK, d), jnp.float32),
          pltpu.VMEM((CHUNK, d), jnp.float32),
          pltpu.VMEM((CHUNK, d), jnp.float32),
          pltpu.SemaphoreType.DMA,
          pltpu.SemaphoreType.DMA,
          pltpu.SemaphoreType.DMA,
          pltpu.SemaphoreType.DMA,
          pltpu.SemaphoreType.DMA,
          pltpu.SemaphoreType.DMA,
          pltpu.VMEM_SHARED((N, d), jnp.float32),
          pltpu.VMEM_SHARED((NACC, d), jnp.float32),
      ],
  )


_agg64 = _make_agg(D_HID)
_agg32 = _make_agg(N_CLS)


_RB = 1024
_GRID = -(-N // _RB)


def _tc1_body(degw_ref, x_ref, w1_ref, hs1_ref, dinv_ref):
  deg = degw_ref[0, :] + degw_ref[1, :] + 1.0
  dinv = lax.rsqrt(deg)[:, None]
  dinv_ref[...] = jnp.broadcast_to(dinv, (dinv.shape[0], DEG_W))
  h = jnp.dot(x_ref[...], w1_ref[...], preferred_element_type=jnp.float32)
  hs1_ref[...] = h * dinv


def _tc1(degw, x, w1):
  return pl.pallas_call(
      _tc1_body,
      grid=(_GRID,),
      in_specs=[
          pl.BlockSpec((NC, _RB), lambda j: (0, j)),
          pl.BlockSpec((_RB, D_IN), lambda j: (j, 0)),
          pl.BlockSpec((D_IN, D_HID), lambda j: (0, 0)),
      ],
      out_specs=[
          pl.BlockSpec((_RB, D_HID), lambda j: (j, 0)),
          pl.BlockSpec((_RB, DEG_W), lambda j: (j, 0)),
      ],
      out_shape=[
          jax.ShapeDtypeStruct((N, D_HID), jnp.float32),
          jax.ShapeDtypeStruct((N, DEG_W), jnp.float32),
      ],
  )(degw, x, w1)


def _tc2_body(acc_ref, hs1_ref, dinv_ref, b1_ref, w2_ref, hs2_ref):
  s = acc_ref[0, :, :] + acc_ref[1, :, :] + hs1_ref[...]
  dinv = dinv_ref[:, 0:1]
  t = s * dinv + b1_ref[...]
  r = jnp.maximum(t, 0.0)
  h2 = jnp.dot(r, w2_ref[...], preferred_element_type=jnp.float32)
  hs2_ref[...] = h2 * dinv


def _tc2(acc1, hs1, dinv, b1, w2):
  return pl.pallas_call(
      _tc2_body,
      grid=(_GRID,),
      in_specs=[
          pl.BlockSpec((NC, _RB, D_HID), lambda j: (0, j, 0)),
          pl.BlockSpec((_RB, D_HID), lambda j: (j, 0)),
          pl.BlockSpec((_RB, DEG_W), lambda j: (j, 0)),
          pl.BlockSpec((1, D_HID), lambda j: (0, 0)),
          pl.BlockSpec((D_HID, N_CLS), lambda j: (0, 0)),
      ],
      out_specs=pl.BlockSpec((_RB, N_CLS), lambda j: (j, 0)),
      out_shape=jax.ShapeDtypeStruct((N, N_CLS), jnp.float32),
  )(acc1, hs1, dinv, b1, w2)


def _tc3_body(acc_ref, hs2_ref, dinv_ref, b2_ref, out_ref):
  s = acc_ref[0, :, :] + acc_ref[1, :, :] + hs2_ref[...]
  t = s * dinv_ref[:, 0:1] + b2_ref[...]
  m = jnp.max(t, axis=1, keepdims=True)
  e = jnp.exp(t - m)
  lse = jnp.log(jnp.sum(e, axis=1, keepdims=True))
  out_ref[...] = t - m - lse


def _tc3(acc2, hs2, dinv, b2):
  return pl.pallas_call(
      _tc3_body,
      grid=(_GRID,),
      in_specs=[
          pl.BlockSpec((NC, _RB, N_CLS), lambda j: (0, j, 0)),
          pl.BlockSpec((_RB, N_CLS), lambda j: (j, 0)),
          pl.BlockSpec((_RB, DEG_W), lambda j: (j, 0)),
          pl.BlockSpec((1, N_CLS), lambda j: (0, 0)),
      ],
      out_specs=pl.BlockSpec((_RB, N_CLS), lambda j: (j, 0)),
      out_shape=jax.ShapeDtypeStruct((N, N_CLS), jnp.float32),
  )(acc2, hs2, dinv, b2)


@jax.jit
def kernel(x, edge_index, W1, b1, W2, b2):
  pad = E_PAD - E
  src = jnp.concatenate([edge_index[0], jnp.zeros((pad,), jnp.int32)])
  dst = jnp.concatenate([edge_index[1], jnp.full((pad,), DUMMY, jnp.int32)])
  src2d = src.reshape(NW * C, CHUNK)
  dst2d = dst.reshape(NW * C, CHUNK)

  ones_w = jnp.ones((CHUNK,), jnp.float32)
  zeros_w = jnp.zeros((NACC,), jnp.float32)
  zeros64 = jnp.zeros((NACC, D_HID), jnp.float32)
  zeros32 = jnp.zeros((NACC, N_CLS), jnp.float32)

  degw = _deg_kernel()(dst2d, ones_w, zeros_w)
  hs1, dinv = _tc1(degw, x, W1)
  acc1 = _agg64()(hs1, src2d, dst2d, zeros64)
  hs2 = _tc2(acc1, hs1, dinv, b1.reshape(1, D_HID), W2)
  acc2 = _agg32()(hs2, src2d, dst2d, zeros32)
  return _tc3(acc2, hs2, dinv, b2.reshape(1, N_CLS))
